# baseline (speedup 1.0000x reference)
import functools

import jax
import jax.numpy as jnp
from jax import lax
from jax.experimental import pallas as pl
from jax.experimental.pallas import tpu as pltpu
from jax.experimental.pallas import tpu_sc as plsc

N_CORES = 2
N_SUBCORES = 16
NW = N_CORES * N_SUBCORES

BLK = 128


@functools.partial(jax.jit, static_argnums=(2,))
def _gather_rows_t(idx_flat, table, dims):
    NB, SEQ, D = dims
    B = NB * SEQ
    b_per_w = B // NW
    nblk = b_per_w // BLK
    b_rows = NB // NW
    DH, DL, SH, SL = D // 8, 8, SEQ // 128, 128
    mesh = plsc.VectorSubcoreMesh(core_axis_name="c", subcore_axis_name="s")

    @functools.partial(
        pl.kernel,
        mesh=mesh,
        out_type=jax.ShapeDtypeStruct((NB, DH, SH, DL, SL), jnp.float32),
        scratch_types=[
            pltpu.VMEM((b_per_w,), jnp.int32),
            pltpu.VMEM((2, BLK, D), jnp.float32),
            pltpu.VMEM((2, DH, DL, SL), jnp.float32),
            pltpu.SemaphoreType.DMA((2,)),
            pltpu.SemaphoreType.DMA((2,)),
        ],
        compiler_params=pltpu.CompilerParams(use_tc_tiling_on_sc=False, needs_layout_passes=False),
    )
    def k(idx_hbm, table_hbm, out_hbm, idx_v, traw, tt, gsem, wsem):
        wid = lax.axis_index("s") * N_CORES + lax.axis_index("c")
        base = wid * b_per_w

        pltpu.sync_copy(idx_hbm.at[pl.ds(base, b_per_w)], idx_v)

        lanes = lax.iota(jnp.int32, 16)
        row_idx = [lanes + 16 * j for j in range(BLK // 16)]

        def fire_gather(g, slot):
            pltpu.async_copy(
                table_hbm.at[idx_v.at[pl.ds(g * BLK, BLK)]],
                traw.at[slot],
                gsem.at[slot],
            )

        def wait_gather(slot):
            pltpu.make_async_copy(
                table_hbm.at[pl.ds(0, BLK)], traw.at[slot], gsem.at[slot]
            ).wait()

        def transpose(slot):
            src = traw.at[slot]
            dst = tt.at[slot]
            for dh in range(DH):
                for dl in range(DL):
                    col = jnp.full((16,), dh * DL + dl, jnp.int32)
                    for j in range(BLK // 16):
                        vals = plsc.load_gather(src, [row_idx[j], col])
                        dst[dh, dl, pl.ds(j * 16, 16)] = vals

        def fire_write(g, slot):
            b = wid * b_rows + g // (SH)
            sh = g % SH
            pltpu.async_copy(
                tt.at[slot], out_hbm.at[b, :, sh], wsem.at[slot]
            )

        def wait_write(slot):
            pltpu.make_async_copy(
                tt.at[slot], out_hbm.at[0, :, 0], wsem.at[slot]
            ).wait()

        def pair(kk, first):
            g0 = kk * 2
            if not first:
                wait_write(0)
            fire_gather(g0, 0)
            if not first:
                wait_gather(1)
                transpose(1)
                fire_write(g0 - 1, 1)
                wait_write(1)
            fire_gather(g0 + 1, 1)
            wait_gather(0)
            transpose(0)
            fire_write(g0, 0)

        pair(0, True)
        lax.fori_loop(1, nblk // 2, lambda kk, u: (pair(kk, False), u)[1], 0)
        wait_gather(1)
        transpose(1)
        fire_write(nblk - 1, 1)
        wait_write(0)
        wait_write(1)

    return k(idx_flat, table)


def kernel(input_ids, attention_mask, table):
    NB, SEQ = input_ids.shape
    D = table.shape[1]
    idx_flat = input_ids.reshape(NB * SEQ).astype(jnp.int32)
    y5 = _gather_rows_t(idx_flat, table, (NB, SEQ, D))
    token_embeddings = y5.transpose(0, 2, 4, 1, 3).reshape(NB, SEQ, D)
    return (input_ids, token_embeddings, attention_mask)


# speedup vs baseline: 1.0383x; 1.0383x over previous
import functools

import jax
import jax.numpy as jnp
from jax import lax
from jax.experimental import pallas as pl
from jax.experimental.pallas import tpu as pltpu
from jax.experimental.pallas import tpu_sc as plsc

N_CORES = 2
N_SUBCORES = 16
NW = N_CORES * N_SUBCORES

BLK = 128


@functools.partial(jax.jit, static_argnums=(2,))
def _gather_rows_t(idx_flat, table, dims):
    NB, SEQ, D = dims
    B = NB * SEQ
    b_per_w = B // NW
    nblk = b_per_w // BLK
    b_rows = NB // NW
    DH, DL, SH, SL = D // 8, 8, SEQ // 128, 128
    mesh = plsc.VectorSubcoreMesh(core_axis_name="c", subcore_axis_name="s")

    @functools.partial(
        pl.kernel,
        mesh=mesh,
        out_type=jax.ShapeDtypeStruct((NB, DH, SH, DL, SL), jnp.float32),
        scratch_types=[
            pltpu.VMEM((b_per_w,), jnp.int32),
            pltpu.VMEM((2, BLK, D), jnp.float32),
            pltpu.VMEM((2, DH, DL, SL), jnp.float32),
            pltpu.SemaphoreType.DMA((2,)),
            pltpu.SemaphoreType.DMA((2,)),
        ],
        compiler_params=pltpu.CompilerParams(use_tc_tiling_on_sc=False, needs_layout_passes=False),
    )
    def k(idx_hbm, table_hbm, out_hbm, idx_v, traw, tt, gsem, wsem):
        wid = lax.axis_index("s") * N_CORES + lax.axis_index("c")
        base = wid * b_per_w

        pltpu.sync_copy(idx_hbm.at[pl.ds(base, b_per_w)], idx_v)

        lanes = lax.iota(jnp.int32, 16)
        row_idx = [lanes + 16 * j for j in range(BLK // 16)]

        def fire_gather(g, slot):
            pltpu.async_copy(
                table_hbm.at[idx_v.at[pl.ds(g * BLK, BLK)]],
                traw.at[slot],
                gsem.at[slot],
            )

        def wait_gather(slot):
            pltpu.make_async_copy(
                table_hbm.at[pl.ds(0, BLK)], traw.at[slot], gsem.at[slot]
            ).wait()

        def transpose(slot):
            src = traw.at[slot]
            dst = tt.at[slot]
            for dh in range(DH):
                for dl in range(DL):
                    col = jnp.full((16,), dh * DL + dl, jnp.int32)
                    vals = [
                        plsc.load_gather(src, [row_idx[j], col])
                        for j in range(BLK // 16)
                    ]
                    for j in range(BLK // 16):
                        dst[dh, dl, pl.ds(j * 16, 16)] = vals[j]

        def fire_write(g, slot):
            b = wid * b_rows + g // (SH)
            sh = g % SH
            pltpu.async_copy(
                tt.at[slot], out_hbm.at[b, :, sh], wsem.at[slot]
            )

        def wait_write(slot):
            pltpu.make_async_copy(
                tt.at[slot], out_hbm.at[0, :, 0], wsem.at[slot]
            ).wait()

        def pair(kk, first):
            g0 = kk * 2
            if not first:
                wait_write(0)
            fire_gather(g0, 0)
            if not first:
                wait_gather(1)
                transpose(1)
                fire_write(g0 - 1, 1)
                wait_write(1)
            fire_gather(g0 + 1, 1)
            wait_gather(0)
            transpose(0)
            fire_write(g0, 0)

        pair(0, True)
        lax.fori_loop(1, nblk // 2, lambda kk, u: (pair(kk, False), u)[1], 0)
        wait_gather(1)
        transpose(1)
        fire_write(nblk - 1, 1)
        wait_write(0)
        wait_write(1)

    return k(idx_flat, table)


def kernel(input_ids, attention_mask, table):
    NB, SEQ = input_ids.shape
    D = table.shape[1]
    idx_flat = input_ids.reshape(NB * SEQ).astype(jnp.int32)
    y5 = _gather_rows_t(idx_flat, table, (NB, SEQ, D))
    token_embeddings = y5.transpose(0, 2, 4, 1, 3).reshape(NB, SEQ, D)
    return (input_ids, token_embeddings, attention_mask)


# speedup vs baseline: 1.2893x; 1.2417x over previous
import functools

import jax
import jax.numpy as jnp
from jax import lax
from jax.experimental import pallas as pl
from jax.experimental.pallas import tpu as pltpu
from jax.experimental.pallas import tpu_sc as plsc

N_CORES = 2
N_SUBCORES = 16
NW = N_CORES * N_SUBCORES

BLK = 128


@functools.partial(jax.jit, static_argnums=(2,))
def _gather_rows_t(idx_flat, table, dims):
    NB, SEQ, D = dims
    B = NB * SEQ
    b_per_w = B // NW
    nblk = b_per_w // BLK
    b_rows = NB // NW
    DH, DL, SH, SL = D // 8, 8, SEQ // 128, 128
    mesh = plsc.VectorSubcoreMesh(core_axis_name="c", subcore_axis_name="s")

    @functools.partial(
        pl.kernel,
        mesh=mesh,
        out_type=jax.ShapeDtypeStruct((NB, DH, SH, DL, SL), jnp.float32),
        scratch_types=[
            pltpu.VMEM((b_per_w,), jnp.int32),
            pltpu.VMEM((2, BLK, D), jnp.float32),
            pltpu.VMEM((2, DH, DL, SL), jnp.float32),
            pltpu.SemaphoreType.DMA((2,)),
            pltpu.SemaphoreType.DMA((2,)),
        ],
        compiler_params=pltpu.CompilerParams(use_tc_tiling_on_sc=False, needs_layout_passes=False),
    )
    def k(idx_hbm, table_hbm, out_hbm, idx_v, traw, tt, gsem, wsem):
        wid = lax.axis_index("s") * N_CORES + lax.axis_index("c")
        base = wid * b_per_w

        pltpu.sync_copy(idx_hbm.at[pl.ds(base, b_per_w)], idx_v)

        lanes = lax.iota(jnp.int32, 16)
        row_idx = [lanes + 16 * j for j in range(BLK // 16)]

        def fire_gather(g, slot):
            pltpu.async_copy(
                table_hbm.at[idx_v.at[pl.ds(g * BLK, BLK)]],
                traw.at[slot],
                gsem.at[slot],
            )

        def wait_gather(slot):
            pltpu.make_async_copy(
                table_hbm.at[pl.ds(0, BLK)], traw.at[slot], gsem.at[slot]
            ).wait()

        def transpose(slot):
            src = traw.at[slot]
            dst = tt.at[slot]

            def dh_body(dh, carry):
                for dl in range(DL):
                    col = jnp.full((16,), dh * DL + dl, jnp.int32)
                    vals = [
                        plsc.load_gather(src, [row_idx[j], col])
                        for j in range(BLK // 16)
                    ]
                    for j in range(BLK // 16):
                        dst[dh, dl, pl.ds(j * 16, 16)] = vals[j]
                return carry

            lax.fori_loop(0, DH, dh_body, 0)

        def fire_write(g, slot):
            b = wid * b_rows + g // (SH)
            sh = g % SH
            pltpu.async_copy(
                tt.at[slot], out_hbm.at[b, :, sh], wsem.at[slot]
            )

        def wait_write(slot):
            pltpu.make_async_copy(
                tt.at[slot], out_hbm.at[0, :, 0], wsem.at[slot]
            ).wait()

        def pair(kk, first):
            g0 = kk * 2
            if not first:
                wait_write(0)
            fire_gather(g0, 0)
            if not first:
                wait_gather(1)
                transpose(1)
                fire_write(g0 - 1, 1)
                wait_write(1)
            fire_gather(g0 + 1, 1)
            wait_gather(0)
            transpose(0)
            fire_write(g0, 0)

        pair(0, True)
        lax.fori_loop(1, nblk // 2, lambda kk, u: (pair(kk, False), u)[1], 0)
        wait_gather(1)
        transpose(1)
        fire_write(nblk - 1, 1)
        wait_write(0)
        wait_write(1)

    return k(idx_flat, table)


def kernel(input_ids, attention_mask, table):
    NB, SEQ = input_ids.shape
    D = table.shape[1]
    idx_flat = input_ids.reshape(NB * SEQ).astype(jnp.int32)
    y5 = _gather_rows_t(idx_flat, table, (NB, SEQ, D))
    token_embeddings = y5.transpose(0, 2, 4, 1, 3).reshape(NB, SEQ, D)
    return (input_ids, token_embeddings, attention_mask)


# speedup vs baseline: 1.3597x; 1.0546x over previous
import functools

import jax
import jax.numpy as jnp
from jax import lax
from jax.experimental import pallas as pl
from jax.experimental.pallas import tpu as pltpu
from jax.experimental.pallas import tpu_sc as plsc

N_CORES = 2
N_SUBCORES = 16
NW = N_CORES * N_SUBCORES

BLK = 128


@functools.partial(jax.jit, static_argnums=(2,))
def _gather_rows_t(idx_flat, table, dims):
    NB, SEQ, D = dims
    B = NB * SEQ
    b_per_w = B // NW
    nblk = b_per_w // BLK
    b_rows = NB // NW
    DH, DL, SH, SL = D // 8, 8, SEQ // 128, 128
    mesh = plsc.VectorSubcoreMesh(core_axis_name="c", subcore_axis_name="s")

    @functools.partial(
        pl.kernel,
        mesh=mesh,
        out_type=jax.ShapeDtypeStruct((NB, DH, SH, DL, SL), jnp.float32),
        scratch_types=[
            pltpu.VMEM((b_per_w,), jnp.int32),
            pltpu.VMEM((2, BLK, D), jnp.float32),
            pltpu.VMEM((2, DH, DL, SL), jnp.float32),
            pltpu.SemaphoreType.DMA((2,)),
            pltpu.SemaphoreType.DMA((2,)),
        ],
        compiler_params=pltpu.CompilerParams(use_tc_tiling_on_sc=False, needs_layout_passes=False),
    )
    def k(idx_hbm, table_hbm, out_hbm, idx_v, traw, tt, gsem, wsem):
        wid = lax.axis_index("s") * N_CORES + lax.axis_index("c")
        base = wid * b_per_w

        pltpu.sync_copy(idx_hbm.at[pl.ds(base, b_per_w)], idx_v)

        lanes = lax.iota(jnp.int32, 16)
        row_idx = [lanes + 16 * j for j in range(BLK // 16)]

        def fire_gather(g, slot):
            pltpu.async_copy(
                table_hbm.at[idx_v.at[pl.ds(g * BLK, BLK)]],
                traw.at[slot],
                gsem.at[slot],
            )

        def wait_gather(slot):
            pltpu.make_async_copy(
                table_hbm.at[pl.ds(0, BLK)], traw.at[slot], gsem.at[slot]
            ).wait()

        def transpose(slot):
            src = traw.at[slot]
            dst = tt.at[slot]

            @plsc.parallel_loop(0, D, unroll=8)
            def _(d):
                dh = d // DL
                dl = lax.rem(d, DL)
                col = jnp.full((16,), d, jnp.int32)
                vals = [
                    plsc.load_gather(src, [row_idx[j], col])
                    for j in range(BLK // 16)
                ]
                for j in range(BLK // 16):
                    dst[dh, dl, pl.ds(j * 16, 16)] = vals[j]

        def fire_write(g, slot):
            b = wid * b_rows + g // (SH)
            sh = g % SH
            pltpu.async_copy(
                tt.at[slot], out_hbm.at[b, :, sh], wsem.at[slot]
            )

        def wait_write(slot):
            pltpu.make_async_copy(
                tt.at[slot], out_hbm.at[0, :, 0], wsem.at[slot]
            ).wait()

        def pair(kk, first):
            g0 = kk * 2
            if not first:
                wait_write(0)
            fire_gather(g0, 0)
            if not first:
                wait_gather(1)
                transpose(1)
                fire_write(g0 - 1, 1)
                wait_write(1)
            fire_gather(g0 + 1, 1)
            wait_gather(0)
            transpose(0)
            fire_write(g0, 0)

        pair(0, True)
        lax.fori_loop(1, nblk // 2, lambda kk, u: (pair(kk, False), u)[1], 0)
        wait_gather(1)
        transpose(1)
        fire_write(nblk - 1, 1)
        wait_write(0)
        wait_write(1)

    return k(idx_flat, table)


def kernel(input_ids, attention_mask, table):
    NB, SEQ = input_ids.shape
    D = table.shape[1]
    idx_flat = input_ids.reshape(NB * SEQ).astype(jnp.int32)
    y5 = _gather_rows_t(idx_flat, table, (NB, SEQ, D))
    token_embeddings = y5.transpose(0, 2, 4, 1, 3).reshape(NB, SEQ, D)
    return (input_ids, token_embeddings, attention_mask)


# speedup vs baseline: 3.3696x; 2.4781x over previous
import functools

import jax
import jax.numpy as jnp
from jax import lax
from jax.experimental import pallas as pl
from jax.experimental.pallas import tpu as pltpu
from jax.experimental.pallas import tpu_sc as plsc

N_CORES = 2
N_SUBCORES = 16
NW = N_CORES * N_SUBCORES

BLK = 128


@functools.partial(jax.jit, static_argnums=(2,))
def _gather_rows_t(idx_flat, table, dims):
    NB, SEQ, D = dims
    B = NB * SEQ
    b_per_w = B // NW
    nblk = b_per_w // BLK
    b_rows = NB // NW
    DH, DL, SH, SL = D // 8, 8, SEQ // 128, 128
    DP = D + 1
    mesh = plsc.VectorSubcoreMesh(core_axis_name="c", subcore_axis_name="s")

    @functools.partial(
        pl.kernel,
        mesh=mesh,
        out_type=jax.ShapeDtypeStruct((NB, DH, SH, DL, SL), jnp.float32),
        scratch_types=[
            pltpu.VMEM((b_per_w,), jnp.int32),
            pltpu.VMEM((2, BLK, D), jnp.float32),
            pltpu.VMEM((BLK, DP), jnp.float32),
            pltpu.VMEM((2, DH, DL, SL), jnp.float32),
            pltpu.SemaphoreType.DMA((2,)),
            pltpu.SemaphoreType.DMA((2,)),
        ],
        compiler_params=pltpu.CompilerParams(use_tc_tiling_on_sc=False, needs_layout_passes=False),
    )
    def k(idx_hbm, table_hbm, out_hbm, idx_v, traw, tpad, tt, gsem, wsem):
        wid = lax.axis_index("s") * N_CORES + lax.axis_index("c")
        base = wid * b_per_w

        pltpu.sync_copy(idx_hbm.at[pl.ds(base, b_per_w)], idx_v)

        lanes = lax.iota(jnp.int32, 16)
        row_idx = [lanes + 16 * j for j in range(BLK // 16)]

        def fire_gather(g, slot):
            pltpu.async_copy(
                table_hbm.at[idx_v.at[pl.ds(g * BLK, BLK)]],
                traw.at[slot],
                gsem.at[slot],
            )

        def wait_gather(slot):
            pltpu.make_async_copy(
                table_hbm.at[pl.ds(0, BLK)], traw.at[slot], gsem.at[slot]
            ).wait()

        def transpose(slot):
            src = traw.at[slot]
            dst = tt.at[slot]

            @plsc.parallel_loop(0, BLK, unroll=4)
            def _(s):
                for kq in range(D // 16):
                    tpad[s, pl.ds(kq * 16, 16)] = src[s, pl.ds(kq * 16, 16)]

            @plsc.parallel_loop(0, D, unroll=8)
            def _(d):
                dh = d // DL
                dl = lax.rem(d, DL)
                col = jnp.full((16,), d, jnp.int32)
                vals = [
                    plsc.load_gather(tpad, [row_idx[j], col])
                    for j in range(BLK // 16)
                ]
                for j in range(BLK // 16):
                    dst[dh, dl, pl.ds(j * 16, 16)] = vals[j]

        def fire_write(g, slot):
            b = wid * b_rows + g // (SH)
            sh = g % SH
            pltpu.async_copy(
                tt.at[slot], out_hbm.at[b, :, sh], wsem.at[slot]
            )

        def wait_write(slot):
            pltpu.make_async_copy(
                tt.at[slot], out_hbm.at[0, :, 0], wsem.at[slot]
            ).wait()

        def pair(kk, first):
            g0 = kk * 2
            if not first:
                wait_write(0)
            fire_gather(g0, 0)
            if not first:
                wait_gather(1)
                transpose(1)
                fire_write(g0 - 1, 1)
                wait_write(1)
            fire_gather(g0 + 1, 1)
            wait_gather(0)
            transpose(0)
            fire_write(g0, 0)

        pair(0, True)
        lax.fori_loop(1, nblk // 2, lambda kk, u: (pair(kk, False), u)[1], 0)
        wait_gather(1)
        transpose(1)
        fire_write(nblk - 1, 1)
        wait_write(0)
        wait_write(1)

    return k(idx_flat, table)


def kernel(input_ids, attention_mask, table):
    NB, SEQ = input_ids.shape
    D = table.shape[1]
    idx_flat = input_ids.reshape(NB * SEQ).astype(jnp.int32)
    y5 = _gather_rows_t(idx_flat, table, (NB, SEQ, D))
    token_embeddings = y5.transpose(0, 2, 4, 1, 3).reshape(NB, SEQ, D)
    return (input_ids, token_embeddings, attention_mask)


# speedup vs baseline: 5.1159x; 1.5183x over previous
import functools

import jax
import jax.numpy as jnp
from jax import lax
from jax.experimental import pallas as pl
from jax.experimental.pallas import tpu as pltpu
from jax.experimental.pallas import tpu_sc as plsc

N_CORES = 2
N_SUBCORES = 16
NW = N_CORES * N_SUBCORES

BLK = 128


@functools.partial(jax.jit, static_argnums=(2,))
def _gather_rows_t(idx_flat, table, dims):
    NB, SEQ, D = dims
    B = NB * SEQ
    b_per_w = B // NW
    nblk = b_per_w // BLK
    b_rows = NB // NW
    DH, DL, SH, SL = D // 8, 8, SEQ // 128, 128
    DP = D + 1
    mesh = plsc.VectorSubcoreMesh(core_axis_name="c", subcore_axis_name="s")

    @functools.partial(
        pl.kernel,
        mesh=mesh,
        out_type=jax.ShapeDtypeStruct((NB, DH, SH, DL, SL), jnp.float32),
        scratch_types=[
            pltpu.VMEM((b_per_w,), jnp.int32),
            pltpu.VMEM((2, BLK, D), jnp.float32),
            pltpu.VMEM((2, DH, DL, SL), jnp.float32),
            pltpu.SemaphoreType.DMA((2,)),
            pltpu.SemaphoreType.DMA((2,)),
        ],
        compiler_params=pltpu.CompilerParams(use_tc_tiling_on_sc=False, needs_layout_passes=False),
    )
    def k(idx_hbm, table_hbm, out_hbm, idx_v, traw, tt, gsem, wsem):
        wid = lax.axis_index("s") * N_CORES + lax.axis_index("c")
        base = wid * b_per_w

        pltpu.sync_copy(idx_hbm.at[pl.ds(base, b_per_w)], idx_v)

        lanes = lax.iota(jnp.int32, 16)
        row_idx = [lanes + 16 * j for j in range(BLK // 16)]

        def fire_gather(g, slot):
            pltpu.async_copy(
                table_hbm.at[idx_v.at[pl.ds(g * BLK, BLK)]],
                traw.at[slot],
                gsem.at[slot],
            )

        def wait_gather(slot):
            pltpu.make_async_copy(
                table_hbm.at[pl.ds(0, BLK)], traw.at[slot], gsem.at[slot]
            ).wait()

        def transpose(slot):
            src = traw.at[slot]
            dst = tt.at[slot]

            @plsc.parallel_loop(0, D, unroll=8)
            def _(d0):
                d_vec = jnp.bitwise_and(d0 + lanes, D - 1)
                dh_vec = lax.shift_right_logical(d_vec, 3)
                dl_vec = jnp.bitwise_and(d_vec, DL - 1)
                for j in range(BLK // 16):
                    vals = plsc.load_gather(src, [row_idx[j], d_vec])
                    plsc.store_scatter(dst, [dh_vec, dl_vec, row_idx[j]], vals)

        def fire_write(g, slot):
            b = wid * b_rows + g // (SH)
            sh = g % SH
            pltpu.async_copy(
                tt.at[slot], out_hbm.at[b, :, sh], wsem.at[slot]
            )

        def wait_write(slot):
            pltpu.make_async_copy(
                tt.at[slot], out_hbm.at[0, :, 0], wsem.at[slot]
            ).wait()

        def pair(kk, first):
            g0 = kk * 2
            if not first:
                wait_write(0)
            fire_gather(g0, 0)
            if not first:
                wait_gather(1)
                transpose(1)
                fire_write(g0 - 1, 1)
                wait_write(1)
            fire_gather(g0 + 1, 1)
            wait_gather(0)
            transpose(0)
            fire_write(g0, 0)

        pair(0, True)
        lax.fori_loop(1, nblk // 2, lambda kk, u: (pair(kk, False), u)[1], 0)
        wait_gather(1)
        transpose(1)
        fire_write(nblk - 1, 1)
        wait_write(0)
        wait_write(1)

    return k(idx_flat, table)


def kernel(input_ids, attention_mask, table):
    NB, SEQ = input_ids.shape
    D = table.shape[1]
    idx_flat = input_ids.reshape(NB * SEQ).astype(jnp.int32)
    y5 = _gather_rows_t(idx_flat, table, (NB, SEQ, D))
    token_embeddings = y5.transpose(0, 2, 4, 1, 3).reshape(NB, SEQ, D)
    return (input_ids, token_embeddings, attention_mask)


# speedup vs baseline: 5.1997x; 1.0164x over previous
import functools

import jax
import jax.numpy as jnp
from jax import lax
from jax.experimental import pallas as pl
from jax.experimental.pallas import tpu as pltpu
from jax.experimental.pallas import tpu_sc as plsc

N_CORES = 2
N_SUBCORES = 16
NW = N_CORES * N_SUBCORES

BLK = 128


@functools.partial(jax.jit, static_argnums=(2,))
def _gather_rows_t(idx_flat, table, dims):
    NB, SEQ, D = dims
    B = NB * SEQ
    b_per_w = B // NW
    nblk = b_per_w // BLK
    b_rows = NB // NW
    DH, DL, SH, SL = D // 8, 8, SEQ // 128, 128
    DP = D + 1
    mesh = plsc.VectorSubcoreMesh(core_axis_name="c", subcore_axis_name="s")

    @functools.partial(
        pl.kernel,
        mesh=mesh,
        out_type=jax.ShapeDtypeStruct((NB, DH, SH, DL, SL), jnp.float32),
        scratch_types=[
            pltpu.VMEM((b_per_w,), jnp.int32),
            pltpu.VMEM((2, BLK, D), jnp.float32),
            pltpu.VMEM((2, DH, DL, SL), jnp.float32),
            pltpu.SemaphoreType.DMA((2,)),
            pltpu.SemaphoreType.DMA((2,)),
        ],
        compiler_params=pltpu.CompilerParams(use_tc_tiling_on_sc=False, needs_layout_passes=False),
    )
    def k(idx_hbm, table_hbm, out_hbm, idx_v, traw, tt, gsem, wsem):
        wid = lax.axis_index("s") * N_CORES + lax.axis_index("c")
        base = wid * b_per_w

        pltpu.sync_copy(idx_hbm.at[pl.ds(base, b_per_w)], idx_v)

        lanes = lax.iota(jnp.int32, 16)
        row_idx = [lanes + 16 * j for j in range(BLK // 16)]

        def fire_gather(g, slot):
            pltpu.async_copy(
                table_hbm.at[idx_v.at[pl.ds(g * BLK, BLK)]],
                traw.at[slot],
                gsem.at[slot],
            )

        def wait_gather(slot):
            pltpu.make_async_copy(
                table_hbm.at[pl.ds(0, BLK)], traw.at[slot], gsem.at[slot]
            ).wait()

        def transpose(slot):
            src = traw.at[slot]
            dst = tt.at[slot]

            @plsc.parallel_loop(0, D, unroll=8)
            def _(d0):
                d_vec = jnp.bitwise_and(d0 + lanes, D - 1)
                dh_vec = lax.shift_right_logical(d_vec, 3)
                dl_vec = jnp.bitwise_and(d_vec, DL - 1)
                for j in range(BLK // 16):
                    vals = plsc.load_gather(src, [row_idx[j], d_vec])
                    plsc.store_scatter(dst, [dh_vec, dl_vec, row_idx[j]], vals)

        def fire_write(g, slot):
            b = wid * b_rows + (g // (SH * 8)) * 8 + lax.rem(g, 8)
            sh = lax.rem(g // 8, SH)
            pltpu.async_copy(
                tt.at[slot], out_hbm.at[b, :, sh], wsem.at[slot]
            )

        def wait_write(slot):
            pltpu.make_async_copy(
                tt.at[slot], out_hbm.at[0, :, 0], wsem.at[slot]
            ).wait()

        def pair(kk, first):
            g0 = kk * 2
            if not first:
                wait_write(0)
            fire_gather(g0, 0)
            if not first:
                wait_gather(1)
                transpose(1)
                fire_write(g0 - 1, 1)
                wait_write(1)
            fire_gather(g0 + 1, 1)
            wait_gather(0)
            transpose(0)
            fire_write(g0, 0)

        pair(0, True)
        lax.fori_loop(1, nblk // 2, lambda kk, u: (pair(kk, False), u)[1], 0)
        wait_gather(1)
        transpose(1)
        fire_write(nblk - 1, 1)
        wait_write(0)
        wait_write(1)

    return k(idx_flat, table)


def kernel(input_ids, attention_mask, table):
    NB, SEQ = input_ids.shape
    D = table.shape[1]
    ids4 = input_ids.reshape(NB // 8, 8, SEQ // 128, 128)
    idx_flat = ids4.transpose(0, 2, 1, 3).reshape(NB * SEQ).astype(jnp.int32)
    y5 = _gather_rows_t(idx_flat, table, (NB, SEQ, D))
    token_embeddings = y5.transpose(0, 2, 4, 1, 3).reshape(NB, SEQ, D)
    return (input_ids, token_embeddings, attention_mask)


# speedup vs baseline: 5.7729x; 1.1102x over previous
import functools

import jax
import jax.numpy as jnp
from jax import lax
from jax.experimental import pallas as pl
from jax.experimental.pallas import tpu as pltpu
from jax.experimental.pallas import tpu_sc as plsc

N_CORES = 2
N_SUBCORES = 16
NW = N_CORES * N_SUBCORES

BLK = 128


@functools.partial(jax.jit, static_argnums=(2,))
def _gather_rows_t(idx_flat, table, dims):
    NB, SEQ, D = dims
    B = NB * SEQ
    b_per_w = B // NW
    nblk = b_per_w // BLK
    b_rows = NB // NW
    DH, DL, SH, SL = D // 8, 8, SEQ // 128, 128
    DP = D + 1
    mesh = plsc.VectorSubcoreMesh(core_axis_name="c", subcore_axis_name="s")

    @functools.partial(
        pl.kernel,
        mesh=mesh,
        out_type=jax.ShapeDtypeStruct((NB, DH, SH, DL, SL), jnp.float32),
        scratch_types=[
            pltpu.VMEM((b_per_w,), jnp.int32),
            pltpu.VMEM((2, BLK, D), jnp.float32),
            pltpu.VMEM((2, DH, DL, SL), jnp.float32),
            pltpu.SemaphoreType.DMA((2,)),
            pltpu.SemaphoreType.DMA((2,)),
        ],
        compiler_params=pltpu.CompilerParams(use_tc_tiling_on_sc=False, needs_layout_passes=False),
    )
    def k(idx_hbm, table_hbm, out_hbm, idx_v, traw, tt, gsem, wsem):
        wid = lax.axis_index("s") * N_CORES + lax.axis_index("c")
        base = wid * b_per_w

        pltpu.sync_copy(idx_hbm.at[pl.ds(base, b_per_w)], idx_v)

        lanes = lax.iota(jnp.int32, 16)
        row_idx = [lanes + 16 * j for j in range(BLK // 16)]

        def fire_gather(g, slot):
            pltpu.async_copy(
                table_hbm.at[idx_v.at[pl.ds(g * BLK, BLK)]],
                traw.at[slot],
                gsem.at[slot],
            )

        def wait_gather(slot):
            pltpu.make_async_copy(
                table_hbm.at[pl.ds(0, BLK)], traw.at[slot], gsem.at[slot]
            ).wait()

        def transpose(slot):
            src = traw.at[slot]
            dst = tt.at[slot]

            @plsc.parallel_loop(0, D, unroll=8)
            def _(d0):
                d_vec = jnp.bitwise_and(d0 + lanes, D - 1)
                dh_vec = lax.shift_right_logical(d_vec, 3)
                dl_vec = jnp.bitwise_and(d_vec, DL - 1)
                for j in range(BLK // 16):
                    vals = plsc.load_gather(src, [row_idx[j], d_vec])
                    plsc.store_scatter(dst, [dh_vec, dl_vec, row_idx[j]], vals)

        def fire_write(g, slot):
            b = wid * b_rows + (g // (SH * 8)) * 8 + lax.rem(g, 8)
            sh = lax.rem(g // 8, SH)
            pltpu.async_copy(
                tt.at[slot], out_hbm.at[b, :, sh], wsem.at[slot]
            )

        def wait_write(slot):
            pltpu.make_async_copy(
                tt.at[slot], out_hbm.at[0, :, 0], wsem.at[slot]
            ).wait()

        def pair(kk, phase):
            g0 = kk * 2
            fire_gather(g0, 0)
            if phase >= 1:
                wait_gather(1)
                if phase >= 2:
                    wait_write(1)
                transpose(1)
                fire_write(g0 - 1, 1)
            fire_gather(g0 + 1, 1)
            wait_gather(0)
            if phase >= 1:
                wait_write(0)
            transpose(0)
            fire_write(g0, 0)

        pair(0, 0)
        pair(1, 1)
        lax.fori_loop(2, nblk // 2, lambda kk, u: (pair(kk, 2), u)[1], 0)
        wait_gather(1)
        wait_write(1)
        transpose(1)
        fire_write(nblk - 1, 1)
        wait_write(0)
        wait_write(1)

    return k(idx_flat, table)


def kernel(input_ids, attention_mask, table):
    NB, SEQ = input_ids.shape
    D = table.shape[1]
    ids4 = input_ids.reshape(NB // 8, 8, SEQ // 128, 128)
    idx_flat = ids4.transpose(0, 2, 1, 3).reshape(NB * SEQ).astype(jnp.int32)
    y5 = _gather_rows_t(idx_flat, table, (NB, SEQ, D))
    token_embeddings = y5.transpose(0, 2, 4, 1, 3).reshape(NB, SEQ, D)
    return (input_ids, token_embeddings, attention_mask)
